# baseline (speedup 1.0000x reference)
import functools

import jax
import jax.numpy as jnp
from jax import lax
from jax.experimental import pallas as pl
from jax.experimental.pallas import tpu as pltpu
from jax.experimental.pallas import tpu_sc as plsc

ALPHA = 40.0
BETA = 0.25
GAMMA = 0.75
HALPHA = ALPHA * 0.5

_info = plsc.get_sparse_core_info()
_NC, _NS, _L = _info.num_cores, _info.num_subcores, _info.num_lanes
_NW = _NC * _NS


def _make_sc_pos(B):
    b_per_w = B // _NW
    mesh = plsc.VectorSubcoreMesh(core_axis_name="c", subcore_axis_name="s")

    @functools.partial(
        pl.kernel,
        out_type=jax.ShapeDtypeStruct((_NW, _L), jnp.float32),
        mesh=mesh,
        scratch_types=[
            pltpu.VMEM((b_per_w,), jnp.float32),
            pltpu.VMEM((_L,), jnp.float32),
            pltpu.VMEM((_L,), jnp.float32),
        ],
    )
    def sc_pos(pos_hbm, omega_hbm, out_hbm, pos_v, om_v, acc_v):
        wid = lax.axis_index("s") * _NC + lax.axis_index("c")
        base = wid * b_per_w
        pltpu.sync_copy(pos_hbm.at[pl.ds(base, b_per_w)], pos_v)
        pltpu.sync_copy(omega_hbm, om_v)
        om = om_v[...]
        acc = jnp.zeros((_L,), jnp.float32)
        for j in range(b_per_w // _L):
            v = pos_v[pl.ds(j * _L, _L)]
            z = ALPHA * (om - v)
            acc = acc + 1.0 / (1.0 + jnp.exp(-z))
        acc_v[...] = acc
        pltpu.sync_copy(acc_v, out_hbm.at[wid])

    return sc_pos


def _tc_body(costh_ref, label_ref, omega_ref, tsum_ref, pos_ref, acc_ref):
    i = pl.program_id(0)
    n = pl.num_programs(0)
    x = costh_ref[...]
    c = HALPHA * omega_ref[0]
    t = jnp.tanh(c - HALPHA * x)

    @pl.when(i == 0)
    def _():
        acc_ref[0] = 0.0

    acc_ref[0] += jnp.sum(t)

    lane = lax.broadcasted_iota(jnp.int32, x.shape, 1)
    onehot = (lane == label_ref[...][:, None]).astype(jnp.float32)
    pos_ref[...] = jnp.sum(x * onehot, axis=1)

    @pl.when(i == n - 1)
    def _():
        tsum_ref[0] = acc_ref[0]


def kernel(costh, label, omega):
    B, C = costh.shape
    BR = 4096
    label_i32 = label.astype(jnp.int32)
    omega_f32 = omega.astype(jnp.float32)
    omega1 = omega_f32.reshape(1)
    omega_vec = jnp.broadcast_to(omega_f32, (_L,))

    t_all, pos = pl.pallas_call(
        _tc_body,
        grid=(B // BR,),
        in_specs=[
            pl.BlockSpec((BR, C), lambda i: (i, 0)),
            pl.BlockSpec((BR,), lambda i: (i,)),
            pl.BlockSpec(memory_space=pltpu.SMEM),
        ],
        out_specs=[
            pl.BlockSpec(memory_space=pltpu.SMEM),
            pl.BlockSpec((BR,), lambda i: (i,)),
        ],
        out_shape=[
            jax.ShapeDtypeStruct((1,), jnp.float32),
            jax.ShapeDtypeStruct((B,), jnp.float32),
        ],
        scratch_shapes=[pltpu.SMEM((1,), jnp.float32)],
    )(costh, label_i32, omega1)

    sc_partials = _make_sc_pos(B)(pos, omega_vec)

    s_pos = jnp.sum(sc_partials)
    s_all = 0.5 * (B * C) + 0.5 * t_all[0]
    pfa = GAMMA * (B - s_pos) / B
    pmiss = BETA * (s_all - s_pos) / (B * (C - 1))
    return pfa + pmiss


# speedup vs baseline: 1.0155x; 1.0155x over previous
import functools

import jax
import jax.numpy as jnp
from jax import lax
from jax.experimental import pallas as pl
from jax.experimental.pallas import tpu as pltpu
from jax.experimental.pallas import tpu_sc as plsc

ALPHA = 40.0
BETA = 0.25
GAMMA = 0.75
HALPHA = ALPHA * 0.5

_info = plsc.get_sparse_core_info()
_NC, _NS, _L = _info.num_cores, _info.num_subcores, _info.num_lanes
_NW = _NC * _NS


def _make_sc_pos(B):
    b_per_w = B // _NW
    mesh = plsc.VectorSubcoreMesh(core_axis_name="c", subcore_axis_name="s")

    @functools.partial(
        pl.kernel,
        out_type=jax.ShapeDtypeStruct((_NW, _L), jnp.float32),
        mesh=mesh,
        scratch_types=[
            pltpu.VMEM((b_per_w,), jnp.float32),
            pltpu.VMEM((_L,), jnp.float32),
            pltpu.VMEM((_L,), jnp.float32),
        ],
    )
    def sc_pos(pos_hbm, omega_hbm, out_hbm, pos_v, om_v, acc_v):
        wid = lax.axis_index("s") * _NC + lax.axis_index("c")
        base = wid * b_per_w
        pltpu.sync_copy(pos_hbm.at[pl.ds(base, b_per_w)], pos_v)
        pltpu.sync_copy(omega_hbm, om_v)
        om = om_v[...]
        acc = jnp.zeros((_L,), jnp.float32)
        for j in range(b_per_w // _L):
            v = pos_v[pl.ds(j * _L, _L)]
            z = ALPHA * (om - v)
            acc = acc + 1.0 / (1.0 + jnp.exp(-z))
        acc_v[...] = acc
        pltpu.sync_copy(acc_v, out_hbm.at[wid])

    return sc_pos


def _tc_body(costh_ref, label_ref, omega_ref, tsum_ref, pos_ref, acc_ref):
    i = pl.program_id(0)
    n = pl.num_programs(0)
    x = costh_ref[...]
    c = HALPHA * omega_ref[0]
    t = jnp.tanh(c - HALPHA * x)

    @pl.when(i == 0)
    def _():
        acc_ref[0] = 0.0

    acc_ref[0] += jnp.sum(t)

    lane = lax.broadcasted_iota(jnp.int32, x.shape, 1)
    onehot = (lane == label_ref[...][:, None]).astype(jnp.float32)
    pos_ref[...] = jnp.sum(x * onehot, axis=1)

    @pl.when(i == n - 1)
    def _():
        tsum_ref[0] = acc_ref[0]


def kernel(costh, label, omega):
    B, C = costh.shape
    BR = 2048
    label_i32 = label.astype(jnp.int32)
    omega_f32 = omega.astype(jnp.float32)
    omega1 = omega_f32.reshape(1)
    omega_vec = jnp.broadcast_to(omega_f32, (_L,))

    t_all, pos = pl.pallas_call(
        _tc_body,
        grid=(B // BR,),
        in_specs=[
            pl.BlockSpec((BR, C), lambda i: (i, 0)),
            pl.BlockSpec((BR,), lambda i: (i,)),
            pl.BlockSpec(memory_space=pltpu.SMEM),
        ],
        out_specs=[
            pl.BlockSpec(memory_space=pltpu.SMEM),
            pl.BlockSpec((BR,), lambda i: (i,)),
        ],
        out_shape=[
            jax.ShapeDtypeStruct((1,), jnp.float32),
            jax.ShapeDtypeStruct((B,), jnp.float32),
        ],
        scratch_shapes=[pltpu.SMEM((1,), jnp.float32)],
    )(costh, label_i32, omega1)

    sc_partials = _make_sc_pos(B)(pos, omega_vec)

    s_pos = jnp.sum(sc_partials)
    s_all = 0.5 * (B * C) + 0.5 * t_all[0]
    pfa = GAMMA * (B - s_pos) / B
    pmiss = BETA * (s_all - s_pos) / (B * (C - 1))
    return pfa + pmiss
